# Initial kernel scaffold; baseline (speedup 1.0000x reference)
import jax
import jax.numpy as jnp
from jax.experimental import pallas as pl


def kernel(xyz, features, w0, b0, g0, bt0, w1, b1, g1, bt1, w2, b2, g2, bt2):
    raise NotImplementedError("write your pallas kernel here")


# speedup vs baseline: 12.2479x; 12.2479x over previous
import functools

import jax
import jax.numpy as jnp
import numpy as np
from jax import lax
from jax.experimental import pallas as pl
from jax.experimental.pallas import tpu as pltpu
from jax.experimental.pallas import tpu_sc as plsc

B, N, S, K = 16, 4096, 512, 32
RAD2 = np.float32(0.2 ** 2)
EPS = np.float32(1e-5)
M_ROWS = B * S * K
ROW_TILE = 2048
N_TILES = M_ROWS // ROW_TILE


def _fps_body(xyz_ref, far0_ref, xb_ref, yb_ref, zb_ref, dmin_ref):
    colio = lax.broadcasted_iota(jnp.int32, (B, N), 1)
    dmin_ref[...] = jnp.full((B, N), 1e10, dtype=jnp.float32)

    def body(i, far):
        xs = xyz_ref[0]
        ys = xyz_ref[1]
        zs = xyz_ref[2]
        onehot = colio == far
        cx = jnp.sum(jnp.where(onehot, xs, 0.0), axis=1, keepdims=True)
        cy = jnp.sum(jnp.where(onehot, ys, 0.0), axis=1, keepdims=True)
        cz = jnp.sum(jnp.where(onehot, zs, 0.0), axis=1, keepdims=True)
        xb_ref[pl.ds(i, 1), :] = jnp.transpose(cx)
        yb_ref[pl.ds(i, 1), :] = jnp.transpose(cy)
        zb_ref[pl.ds(i, 1), :] = jnp.transpose(cz)
        dx = xs - cx
        dy = ys - cy
        dz = zs - cz
        dist = dx * dx + dy * dy + dz * dz
        dmin = dmin_ref[...]
        dmin = jnp.where(dist < dmin, dist, dmin)
        dmin_ref[...] = dmin
        mx = jnp.max(dmin, axis=1, keepdims=True)
        far2 = jnp.min(jnp.where(dmin == mx, colio, N), axis=1,
                       keepdims=True).astype(jnp.int32)
        return far2

    lax.fori_loop(0, S, body, far0_ref[...])


def _fps(xyz_t, far0):
    return pl.pallas_call(
        _fps_body,
        out_shape=[jax.ShapeDtypeStruct((S, B), jnp.float32)] * 3,
        scratch_shapes=[pltpu.VMEM((B, N), jnp.float32)],
    )(xyz_t, far0)


def _mask_body(xyz_ref, cx_ref, cy_ref, cz_ref, mask_ref):
    x3 = xyz_ref[0]
    xs = x3[0:1, :]
    ys = x3[1:2, :]
    zs = x3[2:3, :]
    cx = cx_ref[0]
    cy = cy_ref[0]
    cz = cz_ref[0]
    c3 = jnp.concatenate([cx, cy, cz], axis=1)
    prod = lax.dot_general(c3, x3, (((1,), (0,)), ((), ())),
                           preferred_element_type=jnp.float32)
    d = -2.0 * prod
    d = d + (cx * cx + cy * cy + cz * cz)
    d = d + (xs * xs + ys * ys + zs * zs)
    cc = jnp.where(d <= RAD2, 1, 0).astype(jnp.int32)
    colio = lax.broadcasted_iota(jnp.int32, (128, N), 1)
    sh = 1
    while sh < N:
        cc = cc + jnp.where(colio >= sh, pltpu.roll(cc, sh, 1), 0)
        sh *= 2
    mask_ref[0] = cc


def _ball_mask(xyz_bt, cxb, cyb, czb):
    cspec = pl.BlockSpec((1, 128, 1), lambda b, j: (b, j, 0))
    return pl.pallas_call(
        _mask_body,
        grid=(B, S // 128),
        in_specs=[
            pl.BlockSpec((1, 3, N), lambda b, j: (b, 0, 0)),
            cspec, cspec, cspec,
        ],
        out_specs=pl.BlockSpec((1, 128, N), lambda b, j: (b, j, 0)),
        out_shape=jax.ShapeDtypeStruct((B, S, N), jnp.int32),
    )(xyz_bt, cxb, cyb, czb)


def _q_body(f_ref, xyz_ref, w0a_ref, w0b_ref, b0_ref, q_ref):
    fb = f_ref[0]
    xb = xyz_ref[0]
    q = lax.dot_general(fb, w0b_ref[...], (((0,), (0,)), ((), ())),
                        preferred_element_type=jnp.float32)
    q = q + lax.dot_general(xb, w0a_ref[...], (((1,), (0,)), ((), ())),
                            preferred_element_type=jnp.float32)
    q_ref[0] = q + b0_ref[...]


def _q_kernel(features, xyz, w0a, w0b, b0):
    return pl.pallas_call(
        _q_body,
        grid=(B, N // 512),
        in_specs=[
            pl.BlockSpec((1, 128, 512), lambda b, j: (b, 0, j)),
            pl.BlockSpec((1, 512, 3), lambda b, j: (b, j, 0)),
            pl.BlockSpec((3, 128), lambda b, j: (0, 0)),
            pl.BlockSpec((128, 128), lambda b, j: (0, 0)),
            pl.BlockSpec((1, 128), lambda b, j: (0, 0)),
        ],
        out_specs=pl.BlockSpec((1, 512, 128), lambda b, j: (b, j, 0)),
        out_shape=jax.ShapeDtypeStruct((B, N, 128), jnp.float32),
    )(features, xyz, w0a, w0b, b0)


def _c_body(nxyz_ref, w0a_ref, c_ref):
    xb = nxyz_ref[0]
    c_ref[0] = lax.dot_general(xb, w0a_ref[...], (((1,), (0,)), ((), ())),
                               preferred_element_type=jnp.float32)


def _c_kernel(new_xyz, w0a):
    return pl.pallas_call(
        _c_body,
        grid=(B,),
        in_specs=[
            pl.BlockSpec((1, S, 3), lambda b: (b, 0, 0)),
            pl.BlockSpec((3, 128), lambda b: (0, 0)),
        ],
        out_specs=pl.BlockSpec((1, S, 128), lambda b: (b, 0, 0)),
        out_shape=jax.ShapeDtypeStruct((B, S, 128), jnp.float32),
    )(new_xyz, w0a)


_ROWS_TOTAL = B * S
_NWORKERS = 32
_ROWS_PER_W = _ROWS_TOTAL // _NWORKERS


def _sc_body(cc_hbm, q_hbm, g_hbm, ccrow, idx32, rows, sem):
    wid = lax.axis_index("s") * 2 + lax.axis_index("c")
    zero16 = jnp.zeros((16,), jnp.int32)
    io16 = lax.iota(jnp.int32, 16)
    last = jnp.full((16,), N - 1, jnp.int32)

    def do_row(i, carry):
        r = wid * _ROWS_PER_W + i
        b = r // S
        base = b * N
        pltpu.sync_copy(cc_hbm.at[r], ccrow)
        tot = plsc.load_gather(ccrow, [last])

        for g in range(2):
            target = g * 16 + io16 + 1
            lo = zero16
            hi = jnp.full((16,), N, jnp.int32)
            for _ in range(13):
                mid = lax.shift_right_logical(lo + hi, 1)
                v = plsc.load_gather(ccrow, [jnp.minimum(mid, N - 1)])
                ge = v >= target
                hi = jnp.where(ge, mid, hi)
                lo = jnp.where(ge, lo, mid + 1)
            idx32[pl.ds(g * 16, 16)] = base + jnp.minimum(lo, N - 1)

        first = plsc.load_gather(idx32, [zero16])
        for g in range(2):
            kv = g * 16 + io16
            cur = idx32[pl.ds(g * 16, 16)]
            idx32[pl.ds(g * 16, 16)] = jnp.where(kv < tot, cur, first)

        pltpu.async_copy(q_hbm.at[idx32], rows, sem).wait()
        pltpu.sync_copy(rows, g_hbm.at[r])
        return carry

    lax.fori_loop(0, _ROWS_PER_W, do_row, jnp.int32(0))


def _sc_select_gather(cc2d, qflat):
    mesh = plsc.VectorSubcoreMesh(core_axis_name="c", subcore_axis_name="s")
    fn = functools.partial(
        pl.kernel,
        mesh=mesh,
        out_type=jax.ShapeDtypeStruct((_ROWS_TOTAL, K, 128), jnp.float32),
        scratch_types=[
            pltpu.VMEM((N,), jnp.int32),
            pltpu.VMEM((K,), jnp.int32),
            pltpu.VMEM((K, 128), jnp.float32),
            pltpu.SemaphoreType.DMA,
        ],
        compiler_params=pltpu.CompilerParams(needs_layout_passes=False),
    )(_sc_body)
    return fn(cc2d, qflat)


def _stats1_body(g_ref, c_ref, ssum_ref, ssq_ref):
    @pl.when(pl.program_id(0) == 0)
    def _():
        ssum_ref[...] = jnp.zeros_like(ssum_ref)
        ssq_ref[...] = jnp.zeros_like(ssq_ref)

    g = g_ref[...].reshape(ROW_TILE // K, K, 128)
    c = c_ref[...]
    pre = g - c[:, None, :]
    ssum_ref[0:1, :] += jnp.sum(pre, axis=(0, 1)).reshape(1, 128)
    ssq_ref[0:1, :] += jnp.sum(pre * pre, axis=(0, 1)).reshape(1, 128)


def _stats1(gflat, cflat):
    return pl.pallas_call(
        _stats1_body,
        grid=(N_TILES,),
        in_specs=[
            pl.BlockSpec((ROW_TILE, 128), lambda t: (t, 0)),
            pl.BlockSpec((ROW_TILE // K, 128), lambda t: (t, 0)),
        ],
        out_specs=[
            pl.BlockSpec((8, 128), lambda t: (0, 0)),
            pl.BlockSpec((8, 128), lambda t: (0, 0)),
        ],
        out_shape=[
            jax.ShapeDtypeStruct((8, 128), jnp.float32),
            jax.ShapeDtypeStruct((8, 128), jnp.float32),
        ],
    )(gflat, cflat)


def _layer_body(x_ref, c_ref, s_ref, t_ref, w_ref, b_ref, o_ref,
                ssum_ref, ssq_ref, *, cin, cout, has_c):
    @pl.when(pl.program_id(0) == 0)
    def _():
        ssum_ref[...] = jnp.zeros_like(ssum_ref)
        ssq_ref[...] = jnp.zeros_like(ssq_ref)

    x = x_ref[...]
    if has_c:
        g3 = x.reshape(ROW_TILE // K, K, cin)
        pre = (g3 - c_ref[...][:, None, :]).reshape(ROW_TILE, cin)
    else:
        pre = x
    h = jnp.maximum(pre * s_ref[...] + t_ref[...], 0.0)
    out = lax.dot_general(h, w_ref[...], (((1,), (0,)), ((), ())),
                          preferred_element_type=jnp.float32)
    out = out + b_ref[...]
    o_ref[...] = out
    ssum_ref[0:1, :] += jnp.sum(out, axis=0).reshape(1, cout)
    ssq_ref[0:1, :] += jnp.sum(out * out, axis=0).reshape(1, cout)


def _layer(x, c, s, t, w, b, cin, cout, has_c):
    body = functools.partial(_layer_body, cin=cin, cout=cout, has_c=has_c)
    in_specs = [
        pl.BlockSpec((ROW_TILE, cin), lambda t_: (t_, 0)),
        pl.BlockSpec((ROW_TILE // K, cin), lambda t_: (t_, 0)),
        pl.BlockSpec((1, cin), lambda t_: (0, 0)),
        pl.BlockSpec((1, cin), lambda t_: (0, 0)),
        pl.BlockSpec((cin, cout), lambda t_: (0, 0)),
        pl.BlockSpec((1, cout), lambda t_: (0, 0)),
    ]
    return pl.pallas_call(
        body,
        grid=(N_TILES,),
        in_specs=in_specs,
        out_specs=[
            pl.BlockSpec((ROW_TILE, cout), lambda t_: (t_, 0)),
            pl.BlockSpec((8, cout), lambda t_: (0, 0)),
            pl.BlockSpec((8, cout), lambda t_: (0, 0)),
        ],
        out_shape=[
            jax.ShapeDtypeStruct((M_ROWS, cout), jnp.float32),
            jax.ShapeDtypeStruct((8, cout), jnp.float32),
            jax.ShapeDtypeStruct((8, cout), jnp.float32),
        ],
    )(x, c, s, t, w, b)


def _pool_body(x_ref, s_ref, t_ref, o_ref):
    h = jnp.maximum(x_ref[...] * s_ref[...] + t_ref[...], 0.0)
    h3 = h.reshape(ROW_TILE // K, K, 512)
    o_ref[...] = jnp.max(h3, axis=1)


def _pool(x, s, t):
    return pl.pallas_call(
        _pool_body,
        grid=(N_TILES,),
        in_specs=[
            pl.BlockSpec((ROW_TILE, 512), lambda t_: (t_, 0)),
            pl.BlockSpec((1, 512), lambda t_: (0, 0)),
            pl.BlockSpec((1, 512), lambda t_: (0, 0)),
        ],
        out_specs=pl.BlockSpec((ROW_TILE // K, 512), lambda t_: (t_, 0)),
        out_shape=jax.ShapeDtypeStruct((B * S, 512), jnp.float32),
    )(x, s, t)


def _bn_fold(ssum, ssq, g, bt):
    mean = ssum[0] / M_ROWS
    var = ssq[0] / M_ROWS - mean * mean
    scale = g / jnp.sqrt(var + EPS)
    shift = bt - mean * scale
    return scale.reshape(1, -1), shift.reshape(1, -1)


def kernel(xyz, features, w0, b0, g0, bt0, w1, b1, g1, bt1,
           w2, b2, g2, bt2):
    xyz_t = jnp.transpose(xyz, (2, 0, 1))
    xyz_bt = jnp.transpose(xyz, (0, 2, 1))
    far0 = jax.random.randint(jax.random.key(42), (B,), 0, N,
                              dtype=jnp.int32).reshape(B, 1)
    xb, yb, zb = _fps(xyz_t, far0)
    xbt, ybt, zbt = xb.T, yb.T, zb.T
    new_xyz = jnp.stack([xbt, ybt, zbt], axis=2)

    mask = _ball_mask(xyz_bt, xbt[:, :, None], ybt[:, :, None],
                      zbt[:, :, None])
    q = _q_kernel(features, xyz, w0[:3], w0[3:], b0.reshape(1, 128))
    cmat = _c_kernel(new_xyz, w0[:3])

    g = _sc_select_gather(mask.reshape(B * S, N), q.reshape(B * N, 128))
    gflat = g.reshape(M_ROWS, 128)
    cflat = cmat.reshape(B * S, 128)

    ssum1, ssq1 = _stats1(gflat, cflat)
    s1, t1 = _bn_fold(ssum1, ssq1, g0, bt0)

    pre2, ssum2, ssq2 = _layer(gflat, cflat, s1, t1, w1,
                               b1.reshape(1, 256), 128, 256, True)
    s2, t2 = _bn_fold(ssum2, ssq2, g1, bt1)

    dummy_c = jnp.zeros((B * S, 256), jnp.float32)
    pre3, ssum3, ssq3 = _layer(pre2, dummy_c, s2, t2, w2,
                               b2.reshape(1, 512), 256, 512, False)
    s3, t3 = _bn_fold(ssum3, ssq3, g2, bt2)

    pooled = _pool(pre3, s3, t3)
    new_features = jnp.transpose(pooled.reshape(B, S, 512), (0, 2, 1))
    return new_xyz, new_features
